# baseline (speedup 1.0000x reference)
import functools

import jax
import jax.numpy as jnp
import numpy as np
from jax import lax
from jax.experimental import pallas as pl
from jax.experimental.pallas import tpu as pltpu
from jax.experimental.pallas import tpu_sc as plsc

_NC, _NS = 2, 16
_NW = _NC * _NS
_C = 32
_NCHUNK = 32
_PW = _C * _NCHUNK
_ROWS = _NW * _PW
_D2 = 1536


def _gather_index_table() -> np.ndarray:
    n = 1024
    k = np.arange(n, dtype=np.int64)
    row = np.zeros(n, np.int64)
    col = np.zeros(n, np.int64)
    for b in range(5):
        col |= ((k >> (2 * b)) & 1) << b
        row |= ((k >> (2 * b + 1)) & 1) << b
    mask = row * 32 + col
    pair = mask[0::2] >> 1
    r = np.arange(_ROWS, dtype=np.int64)
    g = (r >> 9) * 512 + pair[r & 511]
    return g.astype(np.int32).reshape(_NW, _NCHUNK, _C)


_IDX_NP = _gather_index_table()


@functools.cache
def _build_zorder_sc():
    mesh = plsc.VectorSubcoreMesh(core_axis_name="c", subcore_axis_name="s")

    @functools.partial(
        pl.kernel,
        mesh=mesh,
        out_type=jax.ShapeDtypeStruct((_ROWS, _D2), jnp.float32),
        scratch_types=[
            pltpu.VMEM((_NCHUNK, _C), jnp.int32),
            pltpu.VMEM((_C, _D2), jnp.float32),
            pltpu.VMEM((_C, _D2), jnp.float32),
            pltpu.SemaphoreType.DMA,
            pltpu.SemaphoreType.DMA,
        ],
    )
    def _zorder_sc(table, idxs, out, idx_v, buf0, buf1, sem0, sem1):
        wid = lax.axis_index("s") * _NC + lax.axis_index("c")
        base = wid * _PW
        pltpu.sync_copy(idxs.at[wid], idx_v)

        def body(i2, carry):
            ch0 = i2 * 2
            cp0 = pltpu.async_copy(table.at[idx_v.at[ch0]], buf0, sem0)
            cp1 = pltpu.async_copy(table.at[idx_v.at[ch0 + 1]], buf1, sem1)
            cp0.wait()
            pltpu.sync_copy(buf0, out.at[pl.ds(base + ch0 * _C, _C)])
            cp1.wait()
            pltpu.sync_copy(buf1, out.at[pl.ds(base + (ch0 + 1) * _C, _C)])
            return carry

        lax.fori_loop(0, _NCHUNK // 2, body, 0)

    return _zorder_sc


def kernel(inputs):
    b, w, h, c = inputs.shape
    flat = inputs.reshape(_ROWS, _D2)
    out = _build_zorder_sc()(flat, jnp.asarray(_IDX_NP))
    return out.reshape(b, w * h, c)


# speedup vs baseline: 1.0024x; 1.0024x over previous
import functools

import jax
import jax.numpy as jnp
import numpy as np
from jax import lax
from jax.experimental import pallas as pl
from jax.experimental.pallas import tpu as pltpu
from jax.experimental.pallas import tpu_sc as plsc

_NC, _NS = 2, 16
_NW = _NC * _NS
_C = 32
_NCHUNK = 32
_PW = _C * _NCHUNK
_ROWS = _NW * _PW
_D2 = 1536


def _gather_index_table() -> np.ndarray:
    n = 1024
    k = np.arange(n, dtype=np.int64)
    row = np.zeros(n, np.int64)
    col = np.zeros(n, np.int64)
    for b in range(5):
        col |= ((k >> (2 * b)) & 1) << b
        row |= ((k >> (2 * b + 1)) & 1) << b
    mask = row * 32 + col
    pair = mask[0::2] >> 1
    r = np.arange(_ROWS, dtype=np.int64)
    g = (r >> 9) * 512 + pair[r & 511]
    return g.astype(np.int32).reshape(_NW, _NCHUNK, _C)


_IDX_NP = _gather_index_table()


@functools.cache
def _build_zorder_sc():
    mesh = plsc.VectorSubcoreMesh(core_axis_name="c", subcore_axis_name="s")

    @functools.partial(
        pl.kernel,
        mesh=mesh,
        out_type=jax.ShapeDtypeStruct((_ROWS, _D2), jnp.float32),
        scratch_types=[
            pltpu.VMEM((_NCHUNK, _C), jnp.int32),
            pltpu.VMEM((_C, _D2), jnp.float32),
            pltpu.VMEM((_C, _D2), jnp.float32),
            pltpu.SemaphoreType.DMA,
            pltpu.SemaphoreType.DMA,
            pltpu.SemaphoreType.DMA,
            pltpu.SemaphoreType.DMA,
        ],
    )
    def _zorder_sc(table, idxs, out, idx_v, buf0, buf1, sg0, sg1, sw0, sw1):
        wid = lax.axis_index("s") * _NC + lax.axis_index("c")
        base = wid * _PW
        pltpu.sync_copy(idxs.at[wid], idx_v)
        bufs, sgs, sws = (buf0, buf1), (sg0, sg1), (sw0, sw1)

        pltpu.async_copy(table.at[idx_v.at[0]], buf0, sg0)
        pltpu.async_copy(table.at[idx_v.at[1]], buf1, sg1)

        def body(i2, carry):
            for b in range(2):
                ch = i2 * 2 + b
                pltpu.make_async_copy(table.at[idx_v.at[ch]], bufs[b], sgs[b]).wait()
                pltpu.async_copy(bufs[b], out.at[pl.ds(base + ch * _C, _C)], sws[b]).wait()
                pltpu.async_copy(table.at[idx_v.at[ch + 2]], bufs[b], sgs[b])
            return carry

        lax.fori_loop(0, _NCHUNK // 2 - 1, body, 0)

        for b in range(2):
            ch = _NCHUNK - 2 + b
            pltpu.make_async_copy(table.at[idx_v.at[ch]], bufs[b], sgs[b]).wait()
            pltpu.async_copy(bufs[b], out.at[pl.ds(base + ch * _C, _C)], sws[b]).wait()

    return _zorder_sc


def kernel(inputs):
    b, w, h, c = inputs.shape
    flat = inputs.reshape(_ROWS, _D2)
    out = _build_zorder_sc()(flat, jnp.asarray(_IDX_NP))
    return out.reshape(b, w * h, c)


# speedup vs baseline: 3.5395x; 3.5310x over previous
import functools

import jax
import jax.numpy as jnp
import numpy as np
from jax import lax
from jax.experimental import pallas as pl
from jax.experimental.pallas import tpu as pltpu
from jax.experimental.pallas import tpu_sc as plsc

_NC, _NS = 2, 16
_NW = _NC * _NS
_C = 64
_NCHUNK = 32
_PW = _C * _NCHUNK
_ROWS = _NW * _PW
_D = 768


def _gather_index_table() -> np.ndarray:
    n = 1024
    k = np.arange(n, dtype=np.int64)
    row = np.zeros(n, np.int64)
    col = np.zeros(n, np.int64)
    for b in range(5):
        col |= ((k >> (2 * b)) & 1) << b
        row |= ((k >> (2 * b + 1)) & 1) << b
    mask = row * 32 + col
    r = np.arange(_ROWS, dtype=np.int64)
    g = (r >> 10) * 1024 + mask[r & 1023]
    return g.astype(np.int32).reshape(_NW, _NCHUNK, _C)


_IDX_NP = _gather_index_table()


@functools.cache
def _build_zorder_sc():
    mesh = plsc.VectorSubcoreMesh(core_axis_name="c", subcore_axis_name="s")

    @functools.partial(
        pl.kernel,
        mesh=mesh,
        out_type=jax.ShapeDtypeStruct((_ROWS, _D), jnp.float32),
        scratch_types=[
            pltpu.VMEM((_NCHUNK, _C), jnp.int32),
            pltpu.VMEM((_C, _D), jnp.float32),
            pltpu.VMEM((_C, _D), jnp.float32),
            pltpu.SemaphoreType.DMA,
            pltpu.SemaphoreType.DMA,
            pltpu.SemaphoreType.DMA,
            pltpu.SemaphoreType.DMA,
        ],
    )
    def _zorder_sc(table, idxs, out, idx_v, buf0, buf1, sg0, sg1, sw0, sw1):
        wid = lax.axis_index("s") * _NC + lax.axis_index("c")
        base = wid * _PW
        pltpu.sync_copy(idxs.at[wid], idx_v)
        bufs, sgs, sws = (buf0, buf1), (sg0, sg1), (sw0, sw1)

        pltpu.async_copy(table.at[idx_v.at[0]], buf0, sg0)
        pltpu.async_copy(table.at[idx_v.at[1]], buf1, sg1)

        def body(i2, carry):
            for b in range(2):
                ch = i2 * 2 + b
                pltpu.make_async_copy(table.at[idx_v.at[ch]], bufs[b], sgs[b]).wait()
                pltpu.async_copy(bufs[b], out.at[pl.ds(base + ch * _C, _C)], sws[b]).wait()
                pltpu.async_copy(table.at[idx_v.at[ch + 2]], bufs[b], sgs[b])
            return carry

        lax.fori_loop(0, _NCHUNK // 2 - 1, body, 0)

        for b in range(2):
            ch = _NCHUNK - 2 + b
            pltpu.make_async_copy(table.at[idx_v.at[ch]], bufs[b], sgs[b]).wait()
            pltpu.async_copy(bufs[b], out.at[pl.ds(base + ch * _C, _C)], sws[b]).wait()

    return _zorder_sc


def kernel(inputs):
    b, w, h, c = inputs.shape
    flat = inputs.reshape(_ROWS, _D)
    out = _build_zorder_sc()(flat, jnp.asarray(_IDX_NP))
    return out.reshape(b, w * h, c)


# speedup vs baseline: 3.5495x; 1.0028x over previous
import functools

import jax
import jax.numpy as jnp
import numpy as np
from jax import lax
from jax.experimental import pallas as pl
from jax.experimental.pallas import tpu as pltpu
from jax.experimental.pallas import tpu_sc as plsc

_NC, _NS = 2, 16
_NW = _NC * _NS
_NB = 4
_C = 32
_NCHUNK = 64
_PW = _C * _NCHUNK
_ROWS = _NW * _PW
_D = 768


def _gather_index_table() -> np.ndarray:
    n = 1024
    k = np.arange(n, dtype=np.int64)
    row = np.zeros(n, np.int64)
    col = np.zeros(n, np.int64)
    for b in range(5):
        col |= ((k >> (2 * b)) & 1) << b
        row |= ((k >> (2 * b + 1)) & 1) << b
    mask = row * 32 + col
    r = np.arange(_ROWS, dtype=np.int64)
    g = (r >> 10) * 1024 + mask[r & 1023]
    return g.astype(np.int32).reshape(_NW, _NCHUNK, _C)


_IDX_NP = _gather_index_table()


@functools.cache
def _build_zorder_sc():
    mesh = plsc.VectorSubcoreMesh(core_axis_name="c", subcore_axis_name="s")

    @functools.partial(
        pl.kernel,
        mesh=mesh,
        out_type=jax.ShapeDtypeStruct((_ROWS, _D), jnp.float32),
        scratch_types=(
            [pltpu.VMEM((_NCHUNK, _C), jnp.int32)]
            + [pltpu.VMEM((_C, _D), jnp.float32)] * _NB
            + [pltpu.SemaphoreType.DMA] * (2 * _NB)
        ),
    )
    def _zorder_sc(table, idxs, out, idx_v, *rest):
        bufs, sgs, sws = rest[:_NB], rest[_NB:2 * _NB], rest[2 * _NB:]
        wid = lax.axis_index("s") * _NC + lax.axis_index("c")
        base = wid * _PW
        pltpu.sync_copy(idxs.at[wid], idx_v)

        for b in range(_NB):
            pltpu.async_copy(table.at[idx_v.at[b]], bufs[b], sgs[b])

        def body(i, carry):
            for b in range(_NB):
                ch = i * _NB + b
                pltpu.make_async_copy(table.at[idx_v.at[ch]], bufs[b], sgs[b]).wait()
                pltpu.async_copy(bufs[b], out.at[pl.ds(base + ch * _C, _C)], sws[b]).wait()
                pltpu.async_copy(table.at[idx_v.at[ch + _NB]], bufs[b], sgs[b])
            return carry

        lax.fori_loop(0, _NCHUNK // _NB - 1, body, 0)

        for b in range(_NB):
            ch = _NCHUNK - _NB + b
            pltpu.make_async_copy(table.at[idx_v.at[ch]], bufs[b], sgs[b]).wait()
            pltpu.async_copy(bufs[b], out.at[pl.ds(base + ch * _C, _C)], sws[b]).wait()

    return _zorder_sc


def kernel(inputs):
    b, w, h, c = inputs.shape
    flat = inputs.reshape(_ROWS, _D)
    out = _build_zorder_sc()(flat, jnp.asarray(_IDX_NP))
    return out.reshape(b, w * h, c)
